# baseline (speedup 1.0000x reference)
import functools

import jax
import jax.numpy as jnp
from jax import lax
from jax.experimental import pallas as pl
from jax.experimental.pallas import tpu as pltpu
from jax.experimental.pallas import tpu_sc as plsc

_NC = 2
_NS = 16
_NW = _NC * _NS
_CHUNK = 128


def _enc_body(x_ref, w_ref, b_ref, o_ref):
    o_ref[:] = (
        jnp.dot(x_ref[:], w_ref[:], preferred_element_type=jnp.float32)
        + b_ref[:]
    )


def _tc_encode(x, W, bvec):
    n, d = x.shape
    h = W.shape[1]
    br = 1000
    return pl.pallas_call(
        _enc_body,
        grid=(n // br,),
        in_specs=[
            pl.BlockSpec((br, d), lambda i: (i, 0)),
            pl.BlockSpec((d, h), lambda i: (0, 0)),
            pl.BlockSpec((1, h), lambda i: (0, 0)),
        ],
        out_specs=pl.BlockSpec((br, h), lambda i: (i, 0)),
        out_shape=jax.ShapeDtypeStruct((n, h), jnp.float32),
    )(x, W, bvec.reshape(1, h))


def _layer_body(h_ref, m_ref, ws_ref, wn_ref, b_ref, g_ref, be_ref, o_ref):
    hv = h_ref[:]
    msg = m_ref[0] + m_ref[1]
    z = (
        jnp.dot(hv, ws_ref[:], preferred_element_type=jnp.float32)
        + jnp.dot(msg, wn_ref[:], preferred_element_type=jnp.float32)
        + b_ref[:]
    )
    mu = jnp.mean(z, axis=1, keepdims=True)
    zc = z - mu
    var = jnp.mean(zc * zc, axis=1, keepdims=True)
    zn = zc * lax.rsqrt(var + 1e-5)
    o_ref[:] = jnp.maximum(zn * g_ref[:] + be_ref[:], 0.0)


def _tc_layer(hm, parts, Ws, Wn, bv, gv, bev):
    n, h = hm.shape
    br = 1000
    return pl.pallas_call(
        _layer_body,
        grid=(n // br,),
        in_specs=[
            pl.BlockSpec((br, h), lambda i: (i, 0)),
            pl.BlockSpec((_NC, br, h), lambda i: (0, i, 0)),
            pl.BlockSpec((h, h), lambda i: (0, 0)),
            pl.BlockSpec((h, h), lambda i: (0, 0)),
            pl.BlockSpec((1, h), lambda i: (0, 0)),
            pl.BlockSpec((1, h), lambda i: (0, 0)),
            pl.BlockSpec((1, h), lambda i: (0, 0)),
        ],
        out_specs=pl.BlockSpec((br, h), lambda i: (i, 0)),
        out_shape=jax.ShapeDtypeStruct((n, h), jnp.float32),
    )(hm, parts, Ws, Wn, bv.reshape(1, h), gv.reshape(1, h), bev.reshape(1, h))


@functools.lru_cache(maxsize=None)
def _sc_segsum(n, h, nct0, nct1):
    acc_rows = -(-(n + 1) // (_NS * _CHUNK)) * (_NS * _CHUNK)
    rps = acc_rows // _NS
    mesh = plsc.VectorSubcoreMesh(core_axis_name="c", subcore_axis_name="s")

    @functools.partial(
        pl.kernel,
        mesh=mesh,
        out_type=jax.ShapeDtypeStruct((_NC, acc_rows, h), jnp.float32),
        scratch_types=[
            pltpu.VMEM((4, 2, _CHUNK), jnp.int32),
            pltpu.VMEM((2, _CHUNK, h), jnp.float32),
            pltpu.VMEM_SHARED((acc_rows, h), jnp.float32),
            pltpu.SemaphoreType.DMA,
            pltpu.SemaphoreType.DMA,
            pltpu.SemaphoreType.DMA,
        ],
    )
    def segsum(h_hbm, idx_hbm, zrow_hbm, out_hbm, idx_v, rows_v, acc,
               sem_i, sem_g, sem_s):
        cid = lax.axis_index("c")
        sid = lax.axis_index("s")
        cnt = jnp.where(cid == 0, nct0, nct1)
        base = jnp.where(cid == 0, sid * nct0, _NS * nct0 + sid * nct1)

        for r in range(rps // _CHUNK):
            pltpu.sync_copy(zrow_hbm,
                            acc.at[pl.ds(sid * rps + r * _CHUNK, _CHUNK)])
        plsc.subcore_barrier()

        def idx_copy(ci, slot):
            return pltpu.make_async_copy(idx_hbm.at[base + ci],
                                         idx_v.at[slot], sem_i)

        def gather(slot, buf):
            return pltpu.make_async_copy(h_hbm.at[idx_v.at[slot, 0]],
                                         rows_v.at[buf], sem_g)

        def scatter_start(slot, buf):
            pltpu.async_copy(rows_v.at[buf], acc.at[idx_v.at[slot, 1]],
                             sem_s, add=True)

        def scatter_wait(slot, buf):
            pltpu.make_async_copy(rows_v.at[buf], acc.at[idx_v.at[slot, 1]],
                                  sem_s).wait()

        @pl.when(cnt > 0)
        def _():
            pltpu.sync_copy(idx_hbm.at[base], idx_v.at[0])
            gather(0, 0).start()
            idx_copy(1, 1).start()

        def step(ci, slot, buf):
            nci = ci + 1
            gather(slot, buf).wait()
            scatter_start(slot, buf)

            @pl.when(ci >= 1)
            def _():
                scatter_wait((slot - 1) % 4, buf ^ 1)

            @pl.when(nci < cnt)
            def _():
                idx_copy(nci, (slot + 1) % 4).wait()
                gather((slot + 1) % 4, buf ^ 1).start()

            @pl.when(ci + 2 < cnt)
            def _():
                idx_copy(ci + 2, (slot + 2) % 4).start()

        def outer(g, carry):
            for k in range(4):
                step(g * 4 + k, k, k % 2)
            return carry

        lax.fori_loop(0, cnt // 4, outer, 0)

        @pl.when(cnt > 0)
        def _():
            scatter_wait(3, 1)

        plsc.subcore_barrier()
        pltpu.sync_copy(acc.at[pl.ds(sid * rps, rps)],
                        out_hbm.at[cid, pl.ds(sid * rps, rps)])

    return segsum


def kernel(x, edge_index, W_enc, b_enc, W_self, W_nbr, b, gamma, beta):
    n, d = x.shape
    h = W_enc.shape[1]
    num_layers = W_self.shape[0]
    e = edge_index.shape[1]

    quantum = _NS * _CHUNK * 16
    ep = -(-e // quantum) * quantum
    acc_rows = -(-(n + 1) // (_NS * _CHUNK)) * (_NS * _CHUNK)
    src = edge_index[0]
    dst = edge_index[1]
    if ep > e:
        pad_dst = n + jnp.arange(ep - e, dtype=jnp.int32) % (acc_rows - n)
        src = jnp.concatenate([src, jnp.zeros((ep - e,), jnp.int32)])
        dst = jnp.concatenate([dst, pad_dst])
    tot = ep // _CHUNK
    nct = tot // _NS
    nct0 = nct
    nct1 = nct - nct0
    idx = jnp.stack([src.reshape(tot, _CHUNK),
                     dst.reshape(tot, _CHUNK)], axis=1)
    zrow = jnp.zeros((_CHUNK, h), jnp.float32)

    segsum = _sc_segsum(n, h, nct0, nct1)
    hm = _tc_encode(x, W_enc, b_enc)
    for i in range(num_layers):
        parts = segsum(hm, idx, zrow)
        hm = _tc_layer(hm, parts, W_self[i], W_nbr[i], b[i], gamma[i],
                       beta[i])
    return hm


# speedup vs baseline: 1.1195x; 1.1195x over previous
import functools

import jax
import jax.numpy as jnp
from jax import lax
from jax.experimental import pallas as pl
from jax.experimental.pallas import tpu as pltpu
from jax.experimental.pallas import tpu_sc as plsc

_NC = 2
_NS = 16
_NW = _NC * _NS
_CHUNK = 64
_U = 8


def _enc_body(x_ref, w_ref, b_ref, o_ref):
    o_ref[:] = (
        jnp.dot(x_ref[:], w_ref[:], preferred_element_type=jnp.float32)
        + b_ref[:]
    )


def _tc_encode(x, W, bvec):
    n, d = x.shape
    h = W.shape[1]
    br = 1000
    return pl.pallas_call(
        _enc_body,
        grid=(n // br,),
        in_specs=[
            pl.BlockSpec((br, d), lambda i: (i, 0)),
            pl.BlockSpec((d, h), lambda i: (0, 0)),
            pl.BlockSpec((1, h), lambda i: (0, 0)),
        ],
        out_specs=pl.BlockSpec((br, h), lambda i: (i, 0)),
        out_shape=jax.ShapeDtypeStruct((n, h), jnp.float32),
    )(x, W, bvec.reshape(1, h))


def _layer_body(h_ref, m_ref, ws_ref, wn_ref, b_ref, g_ref, be_ref, o_ref):
    hv = h_ref[:]
    msg = m_ref[0] + m_ref[1]
    z = (
        jnp.dot(hv, ws_ref[:], preferred_element_type=jnp.float32)
        + jnp.dot(msg, wn_ref[:], preferred_element_type=jnp.float32)
        + b_ref[:]
    )
    mu = jnp.mean(z, axis=1, keepdims=True)
    zc = z - mu
    var = jnp.mean(zc * zc, axis=1, keepdims=True)
    zn = zc * lax.rsqrt(var + 1e-5)
    o_ref[:] = jnp.maximum(zn * g_ref[:] + be_ref[:], 0.0)


def _tc_layer(hm, parts, Ws, Wn, bv, gv, bev):
    n, h = hm.shape
    br = 1000
    return pl.pallas_call(
        _layer_body,
        grid=(n // br,),
        in_specs=[
            pl.BlockSpec((br, h), lambda i: (i, 0)),
            pl.BlockSpec((_NC, br, h), lambda i: (0, i, 0)),
            pl.BlockSpec((h, h), lambda i: (0, 0)),
            pl.BlockSpec((h, h), lambda i: (0, 0)),
            pl.BlockSpec((1, h), lambda i: (0, 0)),
            pl.BlockSpec((1, h), lambda i: (0, 0)),
            pl.BlockSpec((1, h), lambda i: (0, 0)),
        ],
        out_specs=pl.BlockSpec((br, h), lambda i: (i, 0)),
        out_shape=jax.ShapeDtypeStruct((n, h), jnp.float32),
    )(hm, parts, Ws, Wn, bv.reshape(1, h), gv.reshape(1, h), bev.reshape(1, h))


@functools.lru_cache(maxsize=None)
def _sc_segsum(n, h, nct0, nct1):
    acc_rows = -(-(n + 1) // (_NS * _CHUNK)) * (_NS * _CHUNK)
    rps = acc_rows // _NS
    mesh = plsc.VectorSubcoreMesh(core_axis_name="c", subcore_axis_name="s")

    @functools.partial(
        pl.kernel,
        mesh=mesh,
        out_type=jax.ShapeDtypeStruct((_NC, acc_rows, h), jnp.float32),
    scratch_types=[
            pltpu.VMEM((_U, 2, _CHUNK), jnp.int32),
            pltpu.VMEM((4, _CHUNK, h), jnp.float32),
            pltpu.VMEM_SHARED((acc_rows, h), jnp.float32),
            pltpu.SemaphoreType.DMA,
            pltpu.SemaphoreType.DMA,
            pltpu.SemaphoreType.DMA,
        ],
    )
    def segsum(h_hbm, idx_hbm, zrow_hbm, out_hbm, idx_v, rows_v, acc,
               sem_i, sem_g, sem_s):
        cid = lax.axis_index("c")
        sid = lax.axis_index("s")
        cnt = jnp.where(cid == 0, nct0, nct1)
        base = jnp.where(cid == 0, sid * nct0, _NS * nct0 + sid * nct1)

        for r in range(rps // _CHUNK):
            pltpu.sync_copy(zrow_hbm,
                            acc.at[pl.ds(sid * rps + r * _CHUNK, _CHUNK)])
        plsc.subcore_barrier()

        def idx_copy(ci, s):
            return pltpu.make_async_copy(idx_hbm.at[base + ci],
                                         idx_v.at[s], sem_i)

        def gather(s, b):
            return pltpu.make_async_copy(h_hbm.at[idx_v.at[s, 0]],
                                         rows_v.at[b], sem_g)

        def scatter_start(s, b):
            pltpu.async_copy(rows_v.at[b], acc.at[idx_v.at[s, 1]],
                             sem_s, add=True)

        def scatter_wait(s, b):
            pltpu.make_async_copy(rows_v.at[b], acc.at[idx_v.at[s, 1]],
                                  sem_s).wait()

        @pl.when(cnt > 0)
        def _():
            pltpu.sync_copy(idx_hbm.at[base], idx_v.at[0])
            gather(0, 0).start()
            pltpu.sync_copy(idx_hbm.at[base + 1], idx_v.at[1])
            gather(1, 1).start()
            idx_copy(2, 2).start()
            idx_copy(3, 3).start()

        def step(ci, k):
            gather(k, k % 4).wait()
            scatter_start(k, k % 4)

            @pl.when(ci >= 2)
            def _():
                scatter_wait((k - 2) % _U, (k - 2) % 4)

            @pl.when(ci + 2 < cnt)
            def _():
                idx_copy(ci + 2, (k + 2) % _U).wait()
                gather((k + 2) % _U, (k + 2) % 4).start()

            @pl.when(ci + 4 < cnt)
            def _():
                idx_copy(ci + 4, (k + 4) % _U).start()

        def outer(g, carry):
            for k in range(_U):
                step(g * _U + k, k)
            return carry

        lax.fori_loop(0, cnt // _U, outer, 0)

        @pl.when(cnt > 0)
        def _():
            scatter_wait(_U - 2, 2)
            scatter_wait(_U - 1, 3)

        plsc.subcore_barrier()
        pltpu.sync_copy(acc.at[pl.ds(sid * rps, rps)],
                        out_hbm.at[cid, pl.ds(sid * rps, rps)])

    return segsum


def kernel(x, edge_index, W_enc, b_enc, W_self, W_nbr, b, gamma, beta):
    n, d = x.shape
    h = W_enc.shape[1]
    num_layers = W_self.shape[0]
    e = edge_index.shape[1]

    quantum = _NS * _CHUNK * 2 * _U
    ep = -(-e // quantum) * quantum
    acc_rows = -(-(n + 1) // (_NS * _CHUNK)) * (_NS * _CHUNK)
    src = edge_index[0]
    dst = edge_index[1]
    if ep > e:
        pad_dst = n + jnp.arange(ep - e, dtype=jnp.int32) % (acc_rows - n)
        src = jnp.concatenate([src, jnp.zeros((ep - e,), jnp.int32)])
        dst = jnp.concatenate([dst, pad_dst])
    tot = ep // _CHUNK
    nct = tot // _NS
    nct0 = nct // 2
    nct1 = nct - nct0
    idx = jnp.stack([src.reshape(tot, _CHUNK),
                     dst.reshape(tot, _CHUNK)], axis=1)
    zrow = jnp.zeros((_CHUNK, h), jnp.float32)

    segsum = _sc_segsum(n, h, nct0, nct1)
    hm = _tc_encode(x, W_enc, b_enc)
    for i in range(num_layers):
        parts = segsum(hm, idx, zrow)
        hm = _tc_layer(hm, parts, W_self[i], W_nbr[i], b[i], gamma[i],
                       beta[i])
    return hm


# speedup vs baseline: 1.1836x; 1.0572x over previous
import functools

import jax
import jax.numpy as jnp
from jax import lax
from jax.experimental import pallas as pl
from jax.experimental.pallas import tpu as pltpu
from jax.experimental.pallas import tpu_sc as plsc

_NC = 2
_NS = 16
_NW = _NC * _NS
_CHUNK = 64
_U = 8


def _enc_body(x_ref, w_ref, b_ref, o_ref):
    o_ref[:] = (
        jnp.dot(x_ref[:], w_ref[:], preferred_element_type=jnp.float32)
        + b_ref[:]
    )


def _tc_encode(x, W, bvec):
    n, d = x.shape
    h = W.shape[1]
    br = 1000
    return pl.pallas_call(
        _enc_body,
        grid=(n // br,),
        in_specs=[
            pl.BlockSpec((br, d), lambda i: (i, 0)),
            pl.BlockSpec((d, h), lambda i: (0, 0)),
            pl.BlockSpec((1, h), lambda i: (0, 0)),
        ],
        out_specs=pl.BlockSpec((br, h), lambda i: (i, 0)),
        out_shape=jax.ShapeDtypeStruct((n, h), jnp.float32),
    )(x, W, bvec.reshape(1, h))


def _layer_body(h_ref, m_ref, ws_ref, wn_ref, b_ref, g_ref, be_ref, o_ref):
    hv = h_ref[:]
    msg = m_ref[0] + m_ref[1]
    z = (
        jnp.dot(hv, ws_ref[:], preferred_element_type=jnp.float32)
        + jnp.dot(msg, wn_ref[:], preferred_element_type=jnp.float32)
        + b_ref[:]
    )
    mu = jnp.mean(z, axis=1, keepdims=True)
    zc = z - mu
    var = jnp.mean(zc * zc, axis=1, keepdims=True)
    zn = zc * lax.rsqrt(var + 1e-5)
    o_ref[:] = jnp.maximum(zn * g_ref[:] + be_ref[:], 0.0)


def _tc_layer(hm, parts, Ws, Wn, bv, gv, bev):
    n, h = hm.shape
    br = 1000
    return pl.pallas_call(
        _layer_body,
        grid=(n // br,),
        in_specs=[
            pl.BlockSpec((br, h), lambda i: (i, 0)),
            pl.BlockSpec((_NC, br, h), lambda i: (0, i, 0)),
            pl.BlockSpec((h, h), lambda i: (0, 0)),
            pl.BlockSpec((h, h), lambda i: (0, 0)),
            pl.BlockSpec((1, h), lambda i: (0, 0)),
            pl.BlockSpec((1, h), lambda i: (0, 0)),
            pl.BlockSpec((1, h), lambda i: (0, 0)),
        ],
        out_specs=pl.BlockSpec((br, h), lambda i: (i, 0)),
        out_shape=jax.ShapeDtypeStruct((n, h), jnp.float32),
    )(hm, parts, Ws, Wn, bv.reshape(1, h), gv.reshape(1, h), bev.reshape(1, h))


@functools.lru_cache(maxsize=None)
def _sc_segsum(n, h, nct0, nct1):
    acc_rows = -(-(n + 1) // (_NS * _CHUNK)) * (_NS * _CHUNK)
    rps = acc_rows // _NS
    mesh = plsc.VectorSubcoreMesh(core_axis_name="c", subcore_axis_name="s")

    @functools.partial(
        pl.kernel,
        mesh=mesh,
        out_type=jax.ShapeDtypeStruct((_NC, acc_rows, h), jnp.float32),
    scratch_types=[
            pltpu.VMEM((_U, 2, _CHUNK), jnp.int32),
            pltpu.VMEM((4, _CHUNK, h), jnp.float32),
            pltpu.VMEM_SHARED((acc_rows, h), jnp.float32),
            pltpu.SemaphoreType.DMA,
            pltpu.SemaphoreType.DMA,
            pltpu.SemaphoreType.DMA,
        ],
    )
    def segsum(h_hbm, idx_hbm, zrow_hbm, out_hbm, idx_v, rows_v, acc,
               sem_i, sem_g, sem_s):
        cid = lax.axis_index("c")
        sid = lax.axis_index("s")
        cnt = jnp.where(cid == 0, nct0, nct1)
        base = jnp.where(cid == 0, sid * nct0, _NS * nct0 + sid * nct1)

        for r in range(rps // _CHUNK):
            pltpu.sync_copy(zrow_hbm,
                            acc.at[pl.ds(sid * rps + r * _CHUNK, _CHUNK)])
        plsc.subcore_barrier()

        def idx_copy(ci, s):
            return pltpu.make_async_copy(idx_hbm.at[base + ci],
                                         idx_v.at[s], sem_i)

        def gather(s, b):
            return pltpu.make_async_copy(h_hbm.at[idx_v.at[s, 0]],
                                         rows_v.at[b], sem_g)

        def scatter_start(s, b):
            pltpu.async_copy(rows_v.at[b], acc.at[idx_v.at[s, 1]],
                             sem_s, add=True)

        def scatter_wait(s, b):
            pltpu.make_async_copy(rows_v.at[b], acc.at[idx_v.at[s, 1]],
                                  sem_s).wait()

        @pl.when(cnt > 0)
        def _():
            pltpu.sync_copy(idx_hbm.at[base], idx_v.at[0])
            gather(0, 0).start()
            pltpu.sync_copy(idx_hbm.at[base + 1], idx_v.at[1])
            gather(1, 1).start()
            idx_copy(2, 2).start()
            idx_copy(3, 3).start()

        def step(ci, k):
            gather(k, k % 4).wait()
            scatter_start(k, k % 4)

            @pl.when(ci >= 2)
            def _():
                scatter_wait((k - 2) % _U, (k - 2) % 4)

            @pl.when(ci + 2 < cnt)
            def _():
                idx_copy(ci + 2, (k + 2) % _U).wait()
                gather((k + 2) % _U, (k + 2) % 4).start()

            @pl.when(ci + 4 < cnt)
            def _():
                idx_copy(ci + 4, (k + 4) % _U).start()

        def outer(g, carry):
            for k in range(_U):
                step(g * _U + k, k)
            return carry

        lax.fori_loop(0, cnt // _U, outer, 0)

        @pl.when(cnt > 0)
        def _():
            scatter_wait(_U - 2, 2)
            scatter_wait(_U - 1, 3)

        plsc.subcore_barrier()
        pltpu.sync_copy(acc.at[pl.ds(sid * rps, rps)],
                        out_hbm.at[cid, pl.ds(sid * rps, rps)])

    return segsum


def kernel(x, edge_index, W_enc, b_enc, W_self, W_nbr, b, gamma, beta):
    n, d = x.shape
    h = W_enc.shape[1]
    num_layers = W_self.shape[0]
    e = edge_index.shape[1]

    quantum = _NS * _CHUNK * 2 * _U
    ep = -(-e // quantum) * quantum
    acc_rows = -(-(n + 1) // (_NS * _CHUNK)) * (_NS * _CHUNK)
    src = edge_index[0]
    dst = edge_index[1]
    if ep > e:
        pad_dst = n + jnp.arange(ep - e, dtype=jnp.int32) % (acc_rows - n)
        src = jnp.concatenate([src, jnp.zeros((ep - e,), jnp.int32)])
        dst = jnp.concatenate([dst, pad_dst])
    tot = ep // _CHUNK
    nct = tot // _NS
    nct0 = 3 * nct // 4
    nct1 = nct - nct0
    idx = jnp.stack([src.reshape(tot, _CHUNK),
                     dst.reshape(tot, _CHUNK)], axis=1)
    zrow = jnp.zeros((_CHUNK, h), jnp.float32)

    segsum = _sc_segsum(n, h, nct0, nct1)
    hm = _tc_encode(x, W_enc, b_enc)
    for i in range(num_layers):
        parts = segsum(hm, idx, zrow)
        hm = _tc_layer(hm, parts, W_self[i], W_nbr[i], b[i], gamma[i],
                       beta[i])
    return hm


# speedup vs baseline: 1.2593x; 1.0639x over previous
import functools

import jax
import jax.numpy as jnp
from jax import lax
from jax.experimental import pallas as pl
from jax.experimental.pallas import tpu as pltpu
from jax.experimental.pallas import tpu_sc as plsc

_NC = 2
_NS = 16
_NW = _NC * _NS
_CHUNK = 128


def _enc_body(x_ref, w_ref, b_ref, o_ref):
    o_ref[:] = (
        jnp.dot(x_ref[:], w_ref[:], preferred_element_type=jnp.float32)
        + b_ref[:]
    )


def _tc_encode(x, W, bvec):
    n, d = x.shape
    h = W.shape[1]
    br = 1000
    return pl.pallas_call(
        _enc_body,
        grid=(n // br,),
        in_specs=[
            pl.BlockSpec((br, d), lambda i: (i, 0)),
            pl.BlockSpec((d, h), lambda i: (0, 0)),
            pl.BlockSpec((1, h), lambda i: (0, 0)),
        ],
        out_specs=pl.BlockSpec((br, h), lambda i: (i, 0)),
        out_shape=jax.ShapeDtypeStruct((n, h), jnp.float32),
    )(x, W, bvec.reshape(1, h))


def _layer_body(h_ref, m_ref, ws_ref, wn_ref, b_ref, g_ref, be_ref, o_ref):
    hv = h_ref[:]
    msg = m_ref[0] + m_ref[1]
    z = (
        jnp.dot(hv, ws_ref[:], preferred_element_type=jnp.float32)
        + jnp.dot(msg, wn_ref[:], preferred_element_type=jnp.float32)
        + b_ref[:]
    )
    mu = jnp.mean(z, axis=1, keepdims=True)
    zc = z - mu
    var = jnp.mean(zc * zc, axis=1, keepdims=True)
    zn = zc * lax.rsqrt(var + 1e-5)
    o_ref[:] = jnp.maximum(zn * g_ref[:] + be_ref[:], 0.0)


def _tc_layer(hm, parts, Ws, Wn, bv, gv, bev):
    n, h = hm.shape
    br = 1000
    return pl.pallas_call(
        _layer_body,
        grid=(n // br,),
        in_specs=[
            pl.BlockSpec((br, h), lambda i: (i, 0)),
            pl.BlockSpec((_NC, br, h), lambda i: (0, i, 0)),
            pl.BlockSpec((h, h), lambda i: (0, 0)),
            pl.BlockSpec((h, h), lambda i: (0, 0)),
            pl.BlockSpec((1, h), lambda i: (0, 0)),
            pl.BlockSpec((1, h), lambda i: (0, 0)),
            pl.BlockSpec((1, h), lambda i: (0, 0)),
        ],
        out_specs=pl.BlockSpec((br, h), lambda i: (i, 0)),
        out_shape=jax.ShapeDtypeStruct((n, h), jnp.float32),
    )(hm, parts, Ws, Wn, bv.reshape(1, h), gv.reshape(1, h), bev.reshape(1, h))


@functools.lru_cache(maxsize=None)
def _sc_segsum(n, h, nct0, nct1):
    acc_rows = -(-(n + 1) // (_NS * _CHUNK)) * (_NS * _CHUNK)
    rps = acc_rows // _NS
    mesh = plsc.VectorSubcoreMesh(core_axis_name="c", subcore_axis_name="s")

    @functools.partial(
        pl.kernel,
        mesh=mesh,
        out_type=jax.ShapeDtypeStruct((_NC, acc_rows, h), jnp.float32),
        scratch_types=[
            pltpu.VMEM((4, 2, _CHUNK), jnp.int32),
            pltpu.VMEM((2, _CHUNK, h), jnp.float32),
            pltpu.VMEM_SHARED((acc_rows, h), jnp.float32),
            pltpu.SemaphoreType.DMA,
            pltpu.SemaphoreType.DMA,
            pltpu.SemaphoreType.DMA,
        ],
    )
    def segsum(h_hbm, idx_hbm, zrow_hbm, out_hbm, idx_v, rows_v, acc,
               sem_i, sem_g, sem_s):
        cid = lax.axis_index("c")
        sid = lax.axis_index("s")
        cnt = jnp.where(cid == 0, nct0, nct1)
        base = jnp.where(cid == 0, sid * nct0, _NS * nct0 + sid * nct1)

        for r in range(rps // _CHUNK):
            pltpu.sync_copy(zrow_hbm,
                            acc.at[pl.ds(sid * rps + r * _CHUNK, _CHUNK)])
        plsc.subcore_barrier()

        def idx_copy(ci, slot):
            return pltpu.make_async_copy(idx_hbm.at[base + ci],
                                         idx_v.at[slot], sem_i)

        def gather(slot, buf):
            return pltpu.make_async_copy(h_hbm.at[idx_v.at[slot, 0]],
                                         rows_v.at[buf], sem_g)

        def scatter_start(slot, buf):
            pltpu.async_copy(rows_v.at[buf], acc.at[idx_v.at[slot, 1]],
                             sem_s, add=True)

        def scatter_wait(slot, buf):
            pltpu.make_async_copy(rows_v.at[buf], acc.at[idx_v.at[slot, 1]],
                                  sem_s).wait()

        pltpu.sync_copy(idx_hbm.at[base], idx_v.at[0])
        gather(0, 0).start()
        idx_copy(1, 1).start()

        def step(ci, slot, buf):
            nci = ci + 1
            gather(slot, buf).wait()
            scatter_start(slot, buf)

            @pl.when(ci >= 1)
            def _():
                scatter_wait((slot - 1) % 4, buf ^ 1)

            @pl.when(nci < cnt)
            def _():
                idx_copy(nci, (slot + 1) % 4).wait()
                gather((slot + 1) % 4, buf ^ 1).start()

            @pl.when(ci + 2 < cnt)
            def _():
                idx_copy(ci + 2, (slot + 2) % 4).start()

        def outer(g, carry):
            for k in range(4):
                step(g * 4 + k, k, k % 2)
            return carry

        lax.fori_loop(0, cnt // 4, outer, 0)
        scatter_wait(3, 1)
        plsc.subcore_barrier()
        pltpu.sync_copy(acc.at[pl.ds(sid * rps, rps)],
                        out_hbm.at[cid, pl.ds(sid * rps, rps)])

    return segsum


def kernel(x, edge_index, W_enc, b_enc, W_self, W_nbr, b, gamma, beta):
    n, d = x.shape
    h = W_enc.shape[1]
    num_layers = W_self.shape[0]
    e = edge_index.shape[1]

    quantum = _NS * _CHUNK * 16
    ep = -(-e // quantum) * quantum
    src = edge_index[0]
    dst = edge_index[1]
    if ep > e:
        src = jnp.concatenate([src, jnp.zeros((ep - e,), jnp.int32)])
        dst = jnp.concatenate([dst, jnp.full((ep - e,), n, jnp.int32)])
    tot = ep // _CHUNK
    nct = tot // _NS
    nct0 = 3 * nct // 4
    nct1 = nct - nct0
    idx = jnp.stack([src.reshape(tot, _CHUNK),
                     dst.reshape(tot, _CHUNK)], axis=1)
    zrow = jnp.zeros((_CHUNK, h), jnp.float32)

    segsum = _sc_segsum(n, h, nct0, nct1)
    hm = _tc_encode(x, W_enc, b_enc)
    for i in range(num_layers):
        parts = segsum(hm, idx, zrow)
        hm = _tc_layer(hm, parts, W_self[i], W_nbr[i], b[i], gamma[i],
                       beta[i])
    return hm


# speedup vs baseline: 1.2767x; 1.0138x over previous
import functools

import jax
import jax.numpy as jnp
from jax import lax
from jax.experimental import pallas as pl
from jax.experimental.pallas import tpu as pltpu
from jax.experimental.pallas import tpu_sc as plsc

_NC = 2
_NS = 16
_NW = _NC * _NS
_CHUNK = 128


def _enc_body(x_ref, w_ref, b_ref, o_ref):
    o_ref[:] = (
        jnp.dot(x_ref[:], w_ref[:], preferred_element_type=jnp.float32)
        + b_ref[:]
    )


def _tc_encode(x, W, bvec):
    n, d = x.shape
    h = W.shape[1]
    br = 1000
    return pl.pallas_call(
        _enc_body,
        grid=(n // br,),
        in_specs=[
            pl.BlockSpec((br, d), lambda i: (i, 0)),
            pl.BlockSpec((d, h), lambda i: (0, 0)),
            pl.BlockSpec((1, h), lambda i: (0, 0)),
        ],
        out_specs=pl.BlockSpec((br, h), lambda i: (i, 0)),
        out_shape=jax.ShapeDtypeStruct((n, h), jnp.float32),
    )(x, W, bvec.reshape(1, h))


def _layer_body(h_ref, m_ref, ws_ref, wn_ref, b_ref, g_ref, be_ref, o_ref):
    hv = h_ref[:]
    msg = m_ref[0] + m_ref[1]
    z = (
        jnp.dot(hv, ws_ref[:], preferred_element_type=jnp.float32)
        + jnp.dot(msg, wn_ref[:], preferred_element_type=jnp.float32)
        + b_ref[:]
    )
    mu = jnp.mean(z, axis=1, keepdims=True)
    zc = z - mu
    var = jnp.mean(zc * zc, axis=1, keepdims=True)
    zn = zc * lax.rsqrt(var + 1e-5)
    o_ref[:] = jnp.maximum(zn * g_ref[:] + be_ref[:], 0.0)


def _tc_layer(hm, parts, Ws, Wn, bv, gv, bev):
    n, h = hm.shape
    br = 1000
    return pl.pallas_call(
        _layer_body,
        grid=(n // br,),
        in_specs=[
            pl.BlockSpec((br, h), lambda i: (i, 0)),
            pl.BlockSpec((_NC, br, h), lambda i: (0, i, 0)),
            pl.BlockSpec((h, h), lambda i: (0, 0)),
            pl.BlockSpec((h, h), lambda i: (0, 0)),
            pl.BlockSpec((1, h), lambda i: (0, 0)),
            pl.BlockSpec((1, h), lambda i: (0, 0)),
            pl.BlockSpec((1, h), lambda i: (0, 0)),
        ],
        out_specs=pl.BlockSpec((br, h), lambda i: (i, 0)),
        out_shape=jax.ShapeDtypeStruct((n, h), jnp.float32),
    )(hm, parts, Ws, Wn, bv.reshape(1, h), gv.reshape(1, h), bev.reshape(1, h))


@functools.lru_cache(maxsize=None)
def _sc_segsum(n, h, nct0, nct1):
    acc_rows = -(-(n + 1) // (_NS * _CHUNK)) * (_NS * _CHUNK)
    rps = acc_rows // _NS
    mesh = plsc.VectorSubcoreMesh(core_axis_name="c", subcore_axis_name="s")

    @functools.partial(
        pl.kernel,
        mesh=mesh,
        out_type=jax.ShapeDtypeStruct((_NC, acc_rows, h), jnp.float32),
        scratch_types=[
            pltpu.VMEM((4, 2, _CHUNK), jnp.int32),
            pltpu.VMEM((2, _CHUNK, h), jnp.float32),
            pltpu.VMEM_SHARED((acc_rows, h), jnp.float32),
            pltpu.SemaphoreType.DMA,
            pltpu.SemaphoreType.DMA,
            pltpu.SemaphoreType.DMA,
        ],
    )
    def segsum(h_hbm, idx_hbm, zrow_hbm, out_hbm, idx_v, rows_v, acc,
               sem_i, sem_g, sem_s):
        cid = lax.axis_index("c")
        sid = lax.axis_index("s")
        cnt = jnp.where(cid == 0, nct0, nct1)
        base = jnp.where(cid == 0, sid * nct0, _NS * nct0 + sid * nct1)

        for r in range(rps // _CHUNK):
            pltpu.sync_copy(zrow_hbm,
                            acc.at[pl.ds(sid * rps + r * _CHUNK, _CHUNK)])
        plsc.subcore_barrier()

        def idx_copy(ci, slot):
            return pltpu.make_async_copy(idx_hbm.at[base + ci],
                                         idx_v.at[slot], sem_i)

        def gather(slot, buf):
            return pltpu.make_async_copy(h_hbm.at[idx_v.at[slot, 0]],
                                         rows_v.at[buf], sem_g)

        def scatter_start(slot, buf):
            pltpu.async_copy(rows_v.at[buf], acc.at[idx_v.at[slot, 1]],
                             sem_s, add=True)

        def scatter_wait(slot, buf):
            pltpu.make_async_copy(rows_v.at[buf], acc.at[idx_v.at[slot, 1]],
                                  sem_s).wait()

        pltpu.sync_copy(idx_hbm.at[base], idx_v.at[0])
        gather(0, 0).start()
        idx_copy(1, 1).start()

        def step(ci, slot, buf):
            nci = ci + 1
            gather(slot, buf).wait()
            scatter_start(slot, buf)

            @pl.when(ci >= 1)
            def _():
                scatter_wait((slot - 1) % 4, buf ^ 1)

            @pl.when(nci < cnt)
            def _():
                idx_copy(nci, (slot + 1) % 4).wait()
                gather((slot + 1) % 4, buf ^ 1).start()

            @pl.when(ci + 2 < cnt)
            def _():
                idx_copy(ci + 2, (slot + 2) % 4).start()

        def outer(g, carry):
            for k in range(4):
                step(g * 4 + k, k, k % 2)
            return carry

        lax.fori_loop(0, cnt // 4, outer, 0)
        scatter_wait(3, 1)
        plsc.subcore_barrier()
        pltpu.sync_copy(acc.at[pl.ds(sid * rps, rps)],
                        out_hbm.at[cid, pl.ds(sid * rps, rps)])

    return segsum


def kernel(x, edge_index, W_enc, b_enc, W_self, W_nbr, b, gamma, beta):
    n, d = x.shape
    h = W_enc.shape[1]
    num_layers = W_self.shape[0]
    e = edge_index.shape[1]

    quantum = _NS * _CHUNK * 16
    ep = -(-e // quantum) * quantum
    src = edge_index[0]
    dst = edge_index[1]
    if ep > e:
        src = jnp.concatenate([src, jnp.zeros((ep - e,), jnp.int32)])
        dst = jnp.concatenate([dst, jnp.full((ep - e,), n, jnp.int32)])
    tot = ep // _CHUNK
    nct = tot // _NS
    nct0 = 4 * nct // 5
    nct1 = nct - nct0
    idx = jnp.stack([src.reshape(tot, _CHUNK),
                     dst.reshape(tot, _CHUNK)], axis=1)
    zrow = jnp.zeros((_CHUNK, h), jnp.float32)

    segsum = _sc_segsum(n, h, nct0, nct1)
    hm = _tc_encode(x, W_enc, b_enc)
    for i in range(num_layers):
        parts = segsum(hm, idx, zrow)
        hm = _tc_layer(hm, parts, W_self[i], W_nbr[i], b[i], gamma[i],
                       beta[i])
    return hm


# speedup vs baseline: 1.3192x; 1.0334x over previous
import functools

import jax
import jax.numpy as jnp
from jax import lax
from jax.experimental import pallas as pl
from jax.experimental.pallas import tpu as pltpu
from jax.experimental.pallas import tpu_sc as plsc

_NC = 2
_NS = 16
_NW = _NC * _NS
_CHUNK = 128


def _enc_body(x_ref, w_ref, b_ref, o_ref):
    o_ref[:] = (
        jnp.dot(x_ref[:], w_ref[:], preferred_element_type=jnp.float32)
        + b_ref[:]
    )


def _tc_encode(x, W, bvec):
    n, d = x.shape
    h = W.shape[1]
    br = 1000
    return pl.pallas_call(
        _enc_body,
        grid=(n // br,),
        in_specs=[
            pl.BlockSpec((br, d), lambda i: (i, 0)),
            pl.BlockSpec((d, h), lambda i: (0, 0)),
            pl.BlockSpec((1, h), lambda i: (0, 0)),
        ],
        out_specs=pl.BlockSpec((br, h), lambda i: (i, 0)),
        out_shape=jax.ShapeDtypeStruct((n, h), jnp.float32),
    )(x, W, bvec.reshape(1, h))


def _layer_body(h_ref, m_ref, ws_ref, wn_ref, b_ref, g_ref, be_ref, o_ref):
    hv = h_ref[:]
    msg = m_ref[0] + m_ref[1]
    z = (
        jnp.dot(hv, ws_ref[:], preferred_element_type=jnp.float32)
        + jnp.dot(msg, wn_ref[:], preferred_element_type=jnp.float32)
        + b_ref[:]
    )
    mu = jnp.mean(z, axis=1, keepdims=True)
    zc = z - mu
    var = jnp.mean(zc * zc, axis=1, keepdims=True)
    zn = zc * lax.rsqrt(var + 1e-5)
    o_ref[:] = jnp.maximum(zn * g_ref[:] + be_ref[:], 0.0)


def _tc_layer(hm, parts, Ws, Wn, bv, gv, bev):
    n, h = hm.shape
    br = 1000
    return pl.pallas_call(
        _layer_body,
        grid=(n // br,),
        in_specs=[
            pl.BlockSpec((br, h), lambda i: (i, 0)),
            pl.BlockSpec((_NC, br, h), lambda i: (0, i, 0)),
            pl.BlockSpec((h, h), lambda i: (0, 0)),
            pl.BlockSpec((h, h), lambda i: (0, 0)),
            pl.BlockSpec((1, h), lambda i: (0, 0)),
            pl.BlockSpec((1, h), lambda i: (0, 0)),
            pl.BlockSpec((1, h), lambda i: (0, 0)),
        ],
        out_specs=pl.BlockSpec((br, h), lambda i: (i, 0)),
        out_shape=jax.ShapeDtypeStruct((n, h), jnp.float32),
    )(hm, parts, Ws, Wn, bv.reshape(1, h), gv.reshape(1, h), bev.reshape(1, h))


@functools.lru_cache(maxsize=None)
def _sc_segsum(n, h, nct0, nct1):
    acc_rows = -(-(n + 1) // (_NS * _CHUNK)) * (_NS * _CHUNK)
    rps = acc_rows // _NS
    mesh = plsc.VectorSubcoreMesh(core_axis_name="c", subcore_axis_name="s")

    @functools.partial(
        pl.kernel,
        mesh=mesh,
        out_type=jax.ShapeDtypeStruct((_NC, acc_rows, h), jnp.float32),
        scratch_types=[
            pltpu.VMEM((4, 2, _CHUNK), jnp.int32),
            pltpu.VMEM((2, _CHUNK, h), jnp.float32),
            pltpu.VMEM_SHARED((acc_rows, h), jnp.float32),
            pltpu.SemaphoreType.DMA,
            pltpu.SemaphoreType.DMA,
            pltpu.SemaphoreType.DMA,
        ],
    )
    def segsum(h_hbm, idx_hbm, zrow_hbm, out_hbm, idx_v, rows_v, acc,
               sem_i, sem_g, sem_s):
        cid = lax.axis_index("c")
        sid = lax.axis_index("s")
        cnt = jnp.where(cid == 0, nct0, nct1)
        base = jnp.where(cid == 0, sid * nct0, _NS * nct0 + sid * nct1)

        for r in range(rps // _CHUNK):
            pltpu.sync_copy(zrow_hbm,
                            acc.at[pl.ds(sid * rps + r * _CHUNK, _CHUNK)])
        plsc.subcore_barrier()

        def idx_copy(ci, slot):
            return pltpu.make_async_copy(idx_hbm.at[base + ci],
                                         idx_v.at[slot], sem_i)

        def gather(slot, buf):
            return pltpu.make_async_copy(h_hbm.at[idx_v.at[slot, 0]],
                                         rows_v.at[buf], sem_g)

        def scatter_start(slot, buf):
            pltpu.async_copy(rows_v.at[buf], acc.at[idx_v.at[slot, 1]],
                             sem_s, add=True)

        def scatter_wait(slot, buf):
            pltpu.make_async_copy(rows_v.at[buf], acc.at[idx_v.at[slot, 1]],
                                  sem_s).wait()

        pltpu.sync_copy(idx_hbm.at[base], idx_v.at[0])
        gather(0, 0).start()
        idx_copy(1, 1).start()

        def step(ci, slot, buf):
            nci = ci + 1
            gather(slot, buf).wait()
            scatter_start(slot, buf)

            @pl.when(ci >= 1)
            def _():
                scatter_wait((slot - 1) % 4, buf ^ 1)

            @pl.when(nci < cnt)
            def _():
                idx_copy(nci, (slot + 1) % 4).wait()
                gather((slot + 1) % 4, buf ^ 1).start()

            @pl.when(ci + 2 < cnt)
            def _():
                idx_copy(ci + 2, (slot + 2) % 4).start()

        def outer(g, carry):
            for k in range(4):
                step(g * 4 + k, k, k % 2)
            return carry

        lax.fori_loop(0, cnt // 4, outer, 0)
        scatter_wait(3, 1)
        plsc.subcore_barrier()
        pltpu.sync_copy(acc.at[pl.ds(sid * rps, rps)],
                        out_hbm.at[cid, pl.ds(sid * rps, rps)])

    return segsum


def kernel(x, edge_index, W_enc, b_enc, W_self, W_nbr, b, gamma, beta):
    n, d = x.shape
    h = W_enc.shape[1]
    num_layers = W_self.shape[0]
    e = edge_index.shape[1]

    quantum = _NS * _CHUNK * 16
    ep = -(-e // quantum) * quantum
    src = edge_index[0]
    dst = edge_index[1]
    if ep > e:
        src = jnp.concatenate([src, jnp.zeros((ep - e,), jnp.int32)])
        dst = jnp.concatenate([dst, jnp.full((ep - e,), n, jnp.int32)])
    tot = ep // _CHUNK
    nct = tot // _NS
    nct0 = (17 * nct // 80) * 4
    nct1 = nct - nct0
    idx = jnp.stack([src.reshape(tot, _CHUNK),
                     dst.reshape(tot, _CHUNK)], axis=1)
    zrow = jnp.zeros((_CHUNK, h), jnp.float32)

    segsum = _sc_segsum(n, h, nct0, nct1)
    hm = _tc_encode(x, W_enc, b_enc)
    for i in range(num_layers):
        parts = segsum(hm, idx, zrow)
        hm = _tc_layer(hm, parts, W_self[i], W_nbr[i], b[i], gamma[i],
                       beta[i])
    return hm


# speedup vs baseline: 1.4076x; 1.0670x over previous
import functools

import jax
import jax.numpy as jnp
from jax import lax
from jax.experimental import pallas as pl
from jax.experimental.pallas import tpu as pltpu
from jax.experimental.pallas import tpu_sc as plsc

_NC = 2
_NS = 16
_NW = _NC * _NS
_CHUNK = 128


def _enc_body(x_ref, w_ref, b_ref, o_ref):
    o_ref[:] = (
        jnp.dot(x_ref[:], w_ref[:], preferred_element_type=jnp.float32)
        + b_ref[:]
    )


def _tc_encode(x, W, bvec):
    n, d = x.shape
    h = W.shape[1]
    br = 1000
    return pl.pallas_call(
        _enc_body,
        grid=(n // br,),
        in_specs=[
            pl.BlockSpec((br, d), lambda i: (i, 0)),
            pl.BlockSpec((d, h), lambda i: (0, 0)),
            pl.BlockSpec((1, h), lambda i: (0, 0)),
        ],
        out_specs=pl.BlockSpec((br, h), lambda i: (i, 0)),
        out_shape=jax.ShapeDtypeStruct((n, h), jnp.float32),
    )(x, W, bvec.reshape(1, h))


def _layer_body(h_ref, m_ref, ws_ref, wn_ref, b_ref, g_ref, be_ref, o_ref):
    hv = h_ref[:]
    msg = m_ref[0] + m_ref[1]
    z = (
        jnp.dot(hv, ws_ref[:], preferred_element_type=jnp.float32)
        + jnp.dot(msg, wn_ref[:], preferred_element_type=jnp.float32)
        + b_ref[:]
    )
    mu = jnp.mean(z, axis=1, keepdims=True)
    zc = z - mu
    var = jnp.mean(zc * zc, axis=1, keepdims=True)
    zn = zc * lax.rsqrt(var + 1e-5)
    o_ref[:] = jnp.maximum(zn * g_ref[:] + be_ref[:], 0.0)


def _tc_layer(hm, parts, Ws, Wn, bv, gv, bev):
    n, h = hm.shape
    br = 1000
    return pl.pallas_call(
        _layer_body,
        grid=(n // br,),
        in_specs=[
            pl.BlockSpec((br, h), lambda i: (i, 0)),
            pl.BlockSpec((_NC, br, h), lambda i: (0, i, 0)),
            pl.BlockSpec((h, h), lambda i: (0, 0)),
            pl.BlockSpec((h, h), lambda i: (0, 0)),
            pl.BlockSpec((1, h), lambda i: (0, 0)),
            pl.BlockSpec((1, h), lambda i: (0, 0)),
            pl.BlockSpec((1, h), lambda i: (0, 0)),
        ],
        out_specs=pl.BlockSpec((br, h), lambda i: (i, 0)),
        out_shape=jax.ShapeDtypeStruct((n, h), jnp.float32),
    )(hm, parts, Ws, Wn, bv.reshape(1, h), gv.reshape(1, h), bev.reshape(1, h))


@functools.lru_cache(maxsize=None)
def _sc_segsum(n, h, nct0, nct1):
    acc_rows = -(-(n + 1) // (_NS * _CHUNK)) * (_NS * _CHUNK)
    rps = acc_rows // _NS
    mesh = plsc.VectorSubcoreMesh(core_axis_name="c", subcore_axis_name="s")

    @functools.partial(
        pl.kernel,
        mesh=mesh,
        out_type=jax.ShapeDtypeStruct((_NC, acc_rows, h), jnp.float32),
        scratch_types=[
            pltpu.VMEM((4, 2, _CHUNK), jnp.int32),
            pltpu.VMEM((2, _CHUNK, h), jnp.float32),
            pltpu.VMEM_SHARED((acc_rows, h), jnp.float32),
            pltpu.SemaphoreType.DMA,
            pltpu.SemaphoreType.DMA,
            pltpu.SemaphoreType.DMA,
        ],
    )
    def segsum(h_hbm, idx_hbm, zrow_hbm, out_hbm, idx_v, rows_v, acc,
               sem_i, sem_g, sem_s):
        cid = lax.axis_index("c")
        sid = lax.axis_index("s")
        cnt = jnp.where(cid == 0, nct0, nct1)
        base = jnp.where(cid == 0, sid * nct0, _NS * nct0 + sid * nct1)

        for r in range(rps // _CHUNK):
            pltpu.sync_copy(zrow_hbm,
                            acc.at[pl.ds(sid * rps + r * _CHUNK, _CHUNK)])
        plsc.subcore_barrier()

        def idx_copy(ci, slot):
            return pltpu.make_async_copy(idx_hbm.at[base + ci],
                                         idx_v.at[slot], sem_i)

        def gather(slot, buf):
            return pltpu.make_async_copy(h_hbm.at[idx_v.at[slot, 0]],
                                         rows_v.at[buf], sem_g)

        def scatter_start(slot, buf):
            pltpu.async_copy(rows_v.at[buf], acc.at[idx_v.at[slot, 1]],
                             sem_s, add=True)

        def scatter_wait(slot, buf):
            pltpu.make_async_copy(rows_v.at[buf], acc.at[idx_v.at[slot, 1]],
                                  sem_s).wait()

        pltpu.sync_copy(idx_hbm.at[base], idx_v.at[0])
        gather(0, 0).start()
        idx_copy(1, 1).start()

        def step(ci, slot, buf):
            nci = ci + 1
            gather(slot, buf).wait()
            scatter_start(slot, buf)

            @pl.when(ci >= 1)
            def _():
                scatter_wait((slot - 1) % 4, buf ^ 1)

            @pl.when(nci < cnt)
            def _():
                idx_copy(nci, (slot + 1) % 4).wait()
                gather((slot + 1) % 4, buf ^ 1).start()

            @pl.when(ci + 2 < cnt)
            def _():
                idx_copy(ci + 2, (slot + 2) % 4).start()

        def outer(g, carry):
            for k in range(4):
                step(g * 4 + k, k, k % 2)
            return carry

        lax.fori_loop(0, cnt // 4, outer, 0)
        scatter_wait(3, 1)
        plsc.subcore_barrier()
        pltpu.sync_copy(acc.at[pl.ds(sid * rps, rps)],
                        out_hbm.at[cid, pl.ds(sid * rps, rps)])

    return segsum


def kernel(x, edge_index, W_enc, b_enc, W_self, W_nbr, b, gamma, beta):
    n, d = x.shape
    h = W_enc.shape[1]
    num_layers = W_self.shape[0]
    e = edge_index.shape[1]

    quantum = _NS * _CHUNK * 16
    ep = -(-e // quantum) * quantum
    src = edge_index[0]
    dst = edge_index[1]
    if ep > e:
        src = jnp.concatenate([src, jnp.zeros((ep - e,), jnp.int32)])
        dst = jnp.concatenate([dst, jnp.full((ep - e,), n, jnp.int32)])
    tot = ep // _CHUNK
    nct = tot // _NS
    nct0 = (9 * nct // 40) * 4
    nct1 = nct - nct0
    idx = jnp.stack([src.reshape(tot, _CHUNK),
                     dst.reshape(tot, _CHUNK)], axis=1)
    zrow = jnp.zeros((_CHUNK, h), jnp.float32)

    segsum = _sc_segsum(n, h, nct0, nct1)
    hm = _tc_encode(x, W_enc, b_enc)
    for i in range(num_layers):
        parts = segsum(hm, idx, zrow)
        hm = _tc_layer(hm, parts, W_self[i], W_nbr[i], b[i], gamma[i],
                       beta[i])
    return hm
